# baseline (speedup 1.0000x reference)
import functools

import jax
import jax.numpy as jnp
from jax.experimental import pallas as pl
from jax.experimental.pallas import tpu as pltpu

NUM_SEG = 128
STAT_W = 136


def _pick_block(n):
    for b in (12800, 6400, 2560, 1280, 640, 320, 160, 80, 40, 16, 8):
        if n % b == 0:
            return b
    return n


def _lrelu(a):
    return jnp.where(a >= 0, a, 0.01 * a)


def _mask_t(ir_ref, blk):
    return (jax.lax.broadcasted_iota(jnp.int32, (NUM_SEG, blk), 0)
            == ir_ref[...]).astype(jnp.bfloat16)


def _mask(ic_ref, blk):
    return (jax.lax.broadcasted_iota(jnp.int32, (blk, NUM_SEG), 1)
            == ic_ref[...]).astype(jnp.bfloat16)


def _dot(a, b):
    return jax.lax.dot_general(a, b, (((1,), (0,)), ((), ())),
                               preferred_element_type=jnp.float32)


def _seg_stats(mt, hb, blk):
    hh = jnp.concatenate(
        [hb, hb * hb, jnp.ones((blk, 8), jnp.bfloat16)], axis=1)
    return _dot(mt, hh)


def _accum_and_finalize(s_ref, st, t_ref, nb):
    g = pl.program_id(0)

    @pl.when(g == 0)
    def _():
        s_ref[...] = st

    @pl.when(g > 0)
    def _():
        s_ref[...] += st

    @pl.when(g == nb - 1)
    def _():
        sums = s_ref[:, 0:64]
        sqs = s_ref[:, 64:128]
        cnt = s_ref[:, 128:129]
        mean = jnp.where(cnt > 0, sums / jnp.maximum(cnt, 1.0), 0.0)
        var = jnp.where(
            cnt > 1, (sqs - cnt * mean * mean) / jnp.maximum(cnt - 1.0, 1.0),
            0.0)
        std = jnp.sqrt(jnp.maximum(var, 0.0))
        rstd = 1.0 / (std + 1e-8)
        table = jnp.concatenate([mean * rstd, rstd], axis=1)
        hi = table.astype(jnp.bfloat16)
        lo = (table - hi.astype(jnp.float32)).astype(jnp.bfloat16)
        t_ref[...] = jnp.concatenate([hi, lo], axis=1)


def _normalize(hb, m, t_ref):
    r2 = _dot(m, t_ref[...])
    r = r2[:, 0:NUM_SEG] + r2[:, NUM_SEG:2 * NUM_SEG]
    return hb.astype(jnp.float32) * r[:, 64:128] - r[:, 0:64]


def _first_kernel(x_ref, ir_ref, w_ref, b_ref, h_ref, t_ref, s_ref, *,
                  blk, nb):
    hb = _lrelu(_dot(x_ref[...], w_ref[...]) + b_ref[...]).astype(jnp.bfloat16)
    h_ref[...] = hb
    _accum_and_finalize(s_ref, _seg_stats(_mask_t(ir_ref, blk), hb, blk),
                        t_ref, nb)


def _mid_kernel(h_ref, ir_ref, ic_ref, t_ref, w_ref, b_ref, ho_ref, to_ref,
                s_ref, *, blk, nb):
    z = _normalize(h_ref[...], _mask(ic_ref, blk), t_ref)
    hb = _lrelu(_dot(z, w_ref[...]) + b_ref[...]).astype(jnp.bfloat16)
    ho_ref[...] = hb
    _accum_and_finalize(s_ref, _seg_stats(_mask_t(ir_ref, blk), hb, blk),
                        to_ref, nb)


def _last_kernel(h_ref, ic_ref, t_ref, w_ref, b_ref, o_ref, *, blk):
    z = _normalize(h_ref[...], _mask(ic_ref, blk), t_ref)
    o_ref[...] = _dot(z, w_ref[...]) + b_ref[...]


def kernel(x, i, W1, b1, W2, b2, W3, b3, W4, b4):
    n, d = x.shape
    blk = _pick_block(n)
    nb = n // blk
    grid = (nb,)
    i_row = i.reshape(1, n)
    i_col = i.reshape(n, 1)

    row_spec = lambda w: pl.BlockSpec((blk, w), lambda g: (g, 0))
    ir_spec = pl.BlockSpec((1, blk), lambda g: (0, g))
    ic_spec = pl.BlockSpec((blk, 1), lambda g: (g, 0))
    full = lambda *s: pl.BlockSpec(s, lambda g: (0,) * len(s))
    tab_shape = jax.ShapeDtypeStruct((NUM_SEG, 2 * NUM_SEG), jnp.bfloat16)
    h_shape = jax.ShapeDtypeStruct((n, 64), jnp.bfloat16)
    scratch = [pltpu.VMEM((NUM_SEG, STAT_W), jnp.float32)]

    h1, t1 = pl.pallas_call(
        functools.partial(_first_kernel, blk=blk, nb=nb),
        grid=grid,
        in_specs=[row_spec(d), ir_spec, full(d, 64), full(1, 64)],
        out_specs=[row_spec(64), full(NUM_SEG, 2 * NUM_SEG)],
        out_shape=[h_shape, tab_shape],
        scratch_shapes=scratch,
    )(x, i_row, W1.T, b1.reshape(1, 64))

    mid = pl.pallas_call(
        functools.partial(_mid_kernel, blk=blk, nb=nb),
        grid=grid,
        in_specs=[row_spec(64), ir_spec, ic_spec, full(NUM_SEG, 2 * NUM_SEG),
                  full(64, 64), full(1, 64)],
        out_specs=[row_spec(64), full(NUM_SEG, 2 * NUM_SEG)],
        out_shape=[h_shape, tab_shape],
        scratch_shapes=scratch,
    )
    h2, t2 = mid(h1, i_row, i_col, t1, W2.T, b2.reshape(1, 64))
    h3, t3 = mid(h2, i_row, i_col, t2, W3.T, b3.reshape(1, 64))

    out = pl.pallas_call(
        functools.partial(_last_kernel, blk=blk),
        grid=grid,
        in_specs=[row_spec(64), ic_spec, full(NUM_SEG, 2 * NUM_SEG),
                  full(64, 3), full(1, 3)],
        out_specs=row_spec(3),
        out_shape=jax.ShapeDtypeStruct((n, 3), jnp.float32),
    )(h3, i_col, t3, W4.T, b4.reshape(1, 3))
    return out


# speedup vs baseline: 1.2033x; 1.2033x over previous
import functools

import jax
import jax.numpy as jnp
from jax.experimental import pallas as pl
from jax.experimental.pallas import tpu as pltpu

NUM_SEG = 128
STAT_W = 136


def _pick_block(n):
    for b in (12800, 6400, 2560, 1280, 640, 320, 160, 80, 40, 16, 8):
        if n % b == 0:
            return b
    return n


def _lrelu(a):
    return jnp.where(a >= 0, a, 0.01 * a)


def _mask_t(ir_ref, blk):
    return (jax.lax.broadcasted_iota(jnp.int32, (NUM_SEG, blk), 0)
            == ir_ref[...]).astype(jnp.bfloat16)


def _dot(a, b):
    return jax.lax.dot_general(a, b, (((1,), (0,)), ((), ())),
                               preferred_element_type=jnp.float32)


def _seg_stats(mt, hb, blk):
    hh = jnp.concatenate(
        [hb, hb * hb, jnp.ones((blk, 8), jnp.bfloat16)], axis=1)
    return _dot(mt, hh)


def _accum_and_finalize(s_ref, st, t_ref, nb):
    g = pl.program_id(0)

    @pl.when(g == 0)
    def _():
        s_ref[...] = st

    @pl.when(g > 0)
    def _():
        s_ref[...] += st

    @pl.when(g == nb - 1)
    def _():
        sums = s_ref[:, 0:64]
        sqs = s_ref[:, 64:128]
        cnt = s_ref[:, 128:129]
        mean = jnp.where(cnt > 0, sums / jnp.maximum(cnt, 1.0), 0.0)
        var = jnp.where(
            cnt > 1, (sqs - cnt * mean * mean) / jnp.maximum(cnt - 1.0, 1.0),
            0.0)
        std = jnp.sqrt(jnp.maximum(var, 0.0))
        rstd = 1.0 / (std + 1e-8)
        table = jnp.concatenate([mean * rstd, rstd], axis=1)
        hi = table.astype(jnp.bfloat16)
        lo = (table - hi.astype(jnp.float32)).astype(jnp.bfloat16)
        t_ref[...] = jnp.concatenate([hi, lo], axis=1)


def _normalize(hb, mt, t_ref):
    r2 = jax.lax.dot_general(mt, t_ref[...], (((0,), (0,)), ((), ())),
                             preferred_element_type=jnp.float32)
    r = r2[:, 0:NUM_SEG] + r2[:, NUM_SEG:2 * NUM_SEG]
    return hb.astype(jnp.float32) * r[:, 64:128] - r[:, 0:64]


def _first_kernel(x_ref, ir_ref, w_ref, b_ref, h_ref, t_ref, s_ref, *,
                  blk, nb):
    hb = _lrelu(_dot(x_ref[...], w_ref[...]) + b_ref[...]).astype(jnp.bfloat16)
    h_ref[...] = hb
    _accum_and_finalize(s_ref, _seg_stats(_mask_t(ir_ref, blk), hb, blk),
                        t_ref, nb)


def _mid_kernel(h_ref, ir_ref, t_ref, w_ref, b_ref, ho_ref, to_ref,
                s_ref, *, blk, nb):
    mt = _mask_t(ir_ref, blk)
    z = _normalize(h_ref[...], mt, t_ref)
    hb = _lrelu(_dot(z, w_ref[...]) + b_ref[...]).astype(jnp.bfloat16)
    ho_ref[...] = hb
    _accum_and_finalize(s_ref, _seg_stats(mt, hb, blk), to_ref, nb)


def _last_kernel(h_ref, ir_ref, t_ref, w_ref, b_ref, o_ref, *, blk):
    z = _normalize(h_ref[...], _mask_t(ir_ref, blk), t_ref)
    o_ref[...] = _dot(z, w_ref[...]) + b_ref[...]


def kernel(x, i, W1, b1, W2, b2, W3, b3, W4, b4):
    n, d = x.shape
    blk = _pick_block(n)
    nb = n // blk
    grid = (nb,)
    i_row = i.reshape(1, n)

    row_spec = lambda w: pl.BlockSpec((blk, w), lambda g: (g, 0))
    ir_spec = pl.BlockSpec((1, blk), lambda g: (0, g))
    full = lambda *s: pl.BlockSpec(s, lambda g: (0,) * len(s))
    tab_shape = jax.ShapeDtypeStruct((NUM_SEG, 2 * NUM_SEG), jnp.bfloat16)
    h_shape = jax.ShapeDtypeStruct((n, 64), jnp.bfloat16)
    scratch = [pltpu.VMEM((NUM_SEG, STAT_W), jnp.float32)]

    h1, t1 = pl.pallas_call(
        functools.partial(_first_kernel, blk=blk, nb=nb),
        grid=grid,
        in_specs=[row_spec(d), ir_spec, full(d, 64), full(1, 64)],
        out_specs=[row_spec(64), full(NUM_SEG, 2 * NUM_SEG)],
        out_shape=[h_shape, tab_shape],
        scratch_shapes=scratch,
    )(x, i_row, W1.T, b1.reshape(1, 64))

    mid = pl.pallas_call(
        functools.partial(_mid_kernel, blk=blk, nb=nb),
        grid=grid,
        in_specs=[row_spec(64), ir_spec, full(NUM_SEG, 2 * NUM_SEG),
                  full(64, 64), full(1, 64)],
        out_specs=[row_spec(64), full(NUM_SEG, 2 * NUM_SEG)],
        out_shape=[h_shape, tab_shape],
        scratch_shapes=scratch,
    )
    h2, t2 = mid(h1, i_row, t1, W2.T, b2.reshape(1, 64))
    h3, t3 = mid(h2, i_row, t2, W3.T, b3.reshape(1, 64))

    out = pl.pallas_call(
        functools.partial(_last_kernel, blk=blk),
        grid=grid,
        in_specs=[row_spec(64), ir_spec, full(NUM_SEG, 2 * NUM_SEG),
                  full(64, 3), full(1, 3)],
        out_specs=row_spec(3),
        out_shape=jax.ShapeDtypeStruct((n, 3), jnp.float32),
    )(h3, i_row, t3, W4.T, b4.reshape(1, 3))
    return out
